# baseline (speedup 1.0000x reference)
import functools

import jax
import jax.numpy as jnp
from jax import lax
from jax.experimental import pallas as pl
from jax.experimental.pallas import tpu as pltpu
from jax.experimental.pallas import tpu_sc as plsc

VOCAB_ = 1000000
EMBED = 64
B_ = 16384
C_ = 20

_NC = 2
_NS = 16
_NW = _NC * _NS
_BPW = B_ // _NW
_CB = 16
_NCHUNK = _BPW // _CB
_G = 4
_NROW = _CB * C_


def _sc_kernel(tgt_tab, ctx_tab, tidx_hbm, cidx_hbm, out_hbm,
               tcid_a, tcid_b, trows_a, trows_b, cidx_a, cidx_b,
               crows_a, crows_b, out_v, accmat_v, sem_a, sem_b):
    wid = lax.axis_index("s") * _NC + lax.axis_index("c")
    base = wid * _BPW
    lane = lax.broadcasted_iota(jnp.int32, (16,), 0)
    sidx = lane * (_G * C_)

    def fire(i, tcid_v, trows_v, cidx_v, crows_v, sem):
        flat = (base + i * _CB) * C_
        pltpu.sync_copy(tidx_hbm.at[pl.ds(base + i * _CB, _CB)], tcid_v)
        pltpu.sync_copy(cidx_hbm.at[pl.ds(flat, _NROW)], cidx_v)
        tv = tcid_v[pl.ds(0, 16)]
        for j in range(_CB):
            pltpu.async_copy(tgt_tab.at[tv[j]], trows_v.at[j], sem)

        def floop(k, _):
            cv = cidx_v[pl.ds(k * 16, 16)]
            for ln in range(16):
                pltpu.async_copy(
                    ctx_tab.at[cv[ln]], crows_v.at[k * 16 + ln], sem)
            return _

        lax.fori_loop(0, _NROW // 16, floop, None)

    def drain(trows_v, crows_v, sem):
        for j in range(_CB):
            pltpu.make_async_copy(tgt_tab.at[0], trows_v.at[j], sem).wait()

        def dloop(k, _):
            for ln in range(16):
                pltpu.make_async_copy(
                    ctx_tab.at[0], crows_v.at[k * 16 + ln], sem).wait()
            return _

        lax.fori_loop(0, _NROW // 16, dloop, None)

    def compute(i, trows_v, crows_v):
        def group_body(g, _):
            for j in range(_G):
                row = g * _G + j
                t = [trows_v[row, pl.ds(16 * m, 16)] for m in range(4)]
                for c in range(C_):
                    rl = j * C_ + c
                    r = g * (_G * C_) + rl
                    acc = crows_v[r, pl.ds(0, 16)] * t[0]
                    acc += crows_v[r, pl.ds(16, 16)] * t[1]
                    acc += crows_v[r, pl.ds(32, 16)] * t[2]
                    acc += crows_v[r, pl.ds(48, 16)] * t[3]
                    plsc.store_scatter(accmat_v, [sidx + rl], acc)
            for k in range(_G * C_ // 16):
                s = accmat_v[pl.ds(16 * k, 16)]
                for m in range(1, 16):
                    s += accmat_v[pl.ds(m * _G * C_ + 16 * k, 16)]
                out_v[pl.ds(g * _G * C_ + 16 * k, 16)] = s
            return _

        lax.fori_loop(0, _CB // _G, group_body, None)
        pltpu.sync_copy(out_v, out_hbm.at[pl.ds((base + i * _CB) * C_, _NROW)])

    fire(0, tcid_a, trows_a, cidx_a, crows_a, sem_a)
    fire(1, tcid_b, trows_b, cidx_b, crows_b, sem_b)

    def pair_body(it, _):
        i = it * 2
        drain(trows_a, crows_a, sem_a)
        compute(i, trows_a, crows_a)
        fire(i + 2, tcid_a, trows_a, cidx_a, crows_a, sem_a)
        drain(trows_b, crows_b, sem_b)
        compute(i + 1, trows_b, crows_b)
        fire(i + 3, tcid_b, trows_b, cidx_b, crows_b, sem_b)
        return _

    lax.fori_loop(0, _NCHUNK // 2 - 1, pair_body, None)
    i = _NCHUNK - 2
    drain(trows_a, crows_a, sem_a)
    compute(i, trows_a, crows_a)
    drain(trows_b, crows_b, sem_b)
    compute(i + 1, trows_b, crows_b)


@jax.jit
def _run(target_words, context_flat, tgt_tab, ctx_tab):
    mesh = plsc.VectorSubcoreMesh(core_axis_name="c", subcore_axis_name="s")
    k = functools.partial(
        pl.kernel,
        mesh=mesh,
        compiler_params=pltpu.CompilerParams(needs_layout_passes=False),
        out_type=jax.ShapeDtypeStruct((B_ * C_,), jnp.float32),
        scratch_types=[
            pltpu.VMEM((_CB,), jnp.int32),
            pltpu.VMEM((_CB,), jnp.int32),
            pltpu.VMEM((_CB, EMBED), jnp.float32),
            pltpu.VMEM((_CB, EMBED), jnp.float32),
            pltpu.VMEM((_NROW,), jnp.int32),
            pltpu.VMEM((_NROW,), jnp.int32),
            pltpu.VMEM((_NROW, EMBED), jnp.float32),
            pltpu.VMEM((_NROW, EMBED), jnp.float32),
            pltpu.VMEM((_NROW,), jnp.float32),
            pltpu.VMEM((16 * _G * C_,), jnp.float32),
            pltpu.SemaphoreType.DMA,
            pltpu.SemaphoreType.DMA,
        ],
    )(_sc_kernel)
    return k(tgt_tab, ctx_tab, target_words, context_flat)


def kernel(target_words, context_words, target_table, context_table):
    context_flat = context_words.reshape(-1)
    return _run(target_words, context_flat, target_table,
                context_table).reshape(B_, C_)
